# baseline (speedup 1.0000x reference)
import jax
import jax.numpy as jnp
from jax.experimental import pallas as pl


def _linear_kernel(x_ref, w_ref, b_ref, o_ref):
    y = jax.lax.dot_general(
        x_ref[...],
        w_ref[...],
        dimension_numbers=(((1,), (1,)), ((), ())),
        preferred_element_type=jnp.float32,
    )
    o_ref[...] = y + b_ref[...]


@jax.jit
def kernel(hidden_states, W0, b0, W1, b1):
    B, S, D = hidden_states.shape
    N = B * S
    x = hidden_states.reshape(N, D)
    TM = 2048

    out = pl.pallas_call(
        _linear_kernel,
        grid=(N // TM,),
        in_specs=[
            pl.BlockSpec((TM, D), lambda i: (i, 0)),
            pl.BlockSpec((D, D), lambda i: (0, 0)),
            pl.BlockSpec((1, D), lambda i: (0, 0)),
        ],
        out_specs=pl.BlockSpec((TM, D), lambda i: (i, 0)),
        out_shape=jax.ShapeDtypeStruct((N, D), jnp.float32),
    )(x, W0, b0.reshape(1, D))
    return out.reshape(B, S, D)
